# baseline (speedup 1.0000x reference)
import functools

import jax
import jax.numpy as jnp
from jax import lax
from jax.experimental import pallas as pl
from jax.experimental.pallas import tpu as pltpu
from jax.experimental.pallas import tpu_sc as plsc


@functools.lru_cache(maxsize=None)
def _make_sc_gather(V, D, B):
    info = plsc.get_sparse_core_info()
    NC, NS, L = info.num_cores, info.num_subcores, info.num_lanes
    NW = NC * NS
    assert D % L == 0 and B % (8 * NW) == 0
    b_per_w = B // NW
    mesh = plsc.VectorSubcoreMesh(core_axis_name="c", subcore_axis_name="s")

    @functools.partial(
        pl.kernel,
        mesh=mesh,
        out_type=jax.ShapeDtypeStruct((B, D), jnp.float32),
        scratch_types=[
            pltpu.VMEM((b_per_w,), jnp.int32),
            pltpu.VMEM((b_per_w, D), jnp.float32),
            pltpu.SemaphoreType.DMA,
        ],
        compiler_params=pltpu.CompilerParams(use_tc_tiling_on_sc=False),
    )
    def sc_gather(table_hbm, idx_hbm, out_hbm, idx_v, rows_v, sem):
        wid = lax.axis_index("s") * NC + lax.axis_index("c")
        base = wid * b_per_w
        pltpu.sync_copy(idx_hbm.at[pl.ds(base, b_per_w)], idx_v)
        pltpu.async_copy(table_hbm.at[idx_v], rows_v, sem).wait()
        pltpu.sync_copy(rows_v, out_hbm.at[pl.ds(base, b_per_w)])

    return sc_gather


def _tc_body(hid_ref, w2_ref, b_ref, out_ref, ls_ref, wt_ref, *, BT, NV, VC):
    i = pl.program_id(0)
    j = pl.program_id(1)
    R = VC // 8

    @pl.when((i == 0) & (j == 0))
    def _unpack():
        def u(k, c):
            x = w2_ref[pl.ds(k * R, R), :]
            wt_ref[k] = x.reshape(R, 8, 16).transpose(2, 0, 1).reshape(16, VC)
            return c

        lax.fori_loop(0, NV, u, 0)

    @pl.when(j == 0)
    def _stats():
        hid = hid_ref[...]

        def step(k, carry):
            m, s = carry
            logits = (
                jnp.dot(hid, wt_ref[k], preferred_element_type=jnp.float32)
                + b_ref[k]
            )
            cm = jnp.max(logits, axis=1, keepdims=True)
            m2 = jnp.maximum(m, cm)
            s2 = s * jnp.exp(m - m2) + jnp.sum(
                jnp.exp(logits - m2), axis=1, keepdims=True
            )
            return m2, s2

        m0 = jnp.full((BT, 1), -1e30, jnp.float32)
        s0 = jnp.zeros((BT, 1), jnp.float32)
        m, s = lax.fori_loop(0, NV, step, (m0, s0))
        ls_ref[...] = m + jnp.log(s)

    logits = (
        jnp.dot(hid_ref[...], wt_ref[j], preferred_element_type=jnp.float32)
        + b_ref[j]
    )
    out_ref[...] = logits - ls_ref[...]


@functools.lru_cache(maxsize=None)
def _make_tc_logsoftmax(B, V, D, BT, VC):
    NV = -(-V // VC)
    VPAD = NV * VC
    W2R = VPAD * D // 128
    body = functools.partial(_tc_body, BT=BT, NV=NV, VC=VC)
    return pl.pallas_call(
        body,
        grid=(B // BT, NV),
        in_specs=[
            pl.BlockSpec((BT, D), lambda i, j: (i, 0)),
            pl.BlockSpec((W2R, 128), lambda i, j: (0, 0)),
            pl.BlockSpec((NV, 1, VC), lambda i, j: (0, 0, 0)),
        ],
        out_specs=pl.BlockSpec((BT, VC), lambda i, j: (i, j)),
        out_shape=jax.ShapeDtypeStruct((B, V), jnp.float32),
        scratch_shapes=[
            pltpu.VMEM((BT, 1), jnp.float32),
            pltpu.VMEM((NV, D, VC), jnp.float32),
        ],
    )


def kernel(center_word_index, emb_table, W, b):
    V, D = emb_table.shape
    (B,) = center_word_index.shape
    BT = 256
    VC = 2048
    NV = -(-V // VC)
    VPAD = NV * VC

    idx = center_word_index.astype(jnp.int32)
    hidden = _make_sc_gather(V, D, B)(emb_table, idx)

    w2 = W.reshape(V * D // 128, 128)
    w2 = jnp.pad(w2, ((0, (VPAD - V) * D // 128), (0, 0)))
    b3 = jnp.pad(b, (0, VPAD - V), constant_values=-1e30).reshape(NV, 1, VC)

    return _make_tc_logsoftmax(B, V, D, BT, VC)(hidden, w2, b3)


# speedup vs baseline: 2.1556x; 2.1556x over previous
import functools

import jax
import jax.numpy as jnp
from jax import lax
from jax.experimental import pallas as pl
from jax.experimental.pallas import tpu as pltpu
from jax.experimental.pallas import tpu_sc as plsc


@functools.lru_cache(maxsize=None)
def _make_sc_gather(V, D, B):
    info = plsc.get_sparse_core_info()
    NC, NS, L = info.num_cores, info.num_subcores, info.num_lanes
    NW = NC * NS
    assert D % L == 0 and B % (8 * NW) == 0
    b_per_w = B // NW
    mesh = plsc.VectorSubcoreMesh(core_axis_name="c", subcore_axis_name="s")

    @functools.partial(
        pl.kernel,
        mesh=mesh,
        out_type=jax.ShapeDtypeStruct((B, D), jnp.float32),
        scratch_types=[
            pltpu.VMEM((b_per_w,), jnp.int32),
            pltpu.VMEM((b_per_w, D), jnp.float32),
            pltpu.SemaphoreType.DMA,
        ],
        compiler_params=pltpu.CompilerParams(use_tc_tiling_on_sc=False),
    )
    def sc_gather(table_hbm, idx_hbm, out_hbm, idx_v, rows_v, sem):
        wid = lax.axis_index("s") * NC + lax.axis_index("c")
        base = wid * b_per_w
        pltpu.sync_copy(idx_hbm.at[pl.ds(base, b_per_w)], idx_v)
        pltpu.async_copy(table_hbm.at[idx_v], rows_v, sem).wait()
        pltpu.sync_copy(rows_v, out_hbm.at[pl.ds(base, b_per_w)])

    return sc_gather


_TN = (((0,), (0,)), ((), ()))


def _tc_body(hid_ref, wtb_all_ref, wtb_blk_ref, out_ref, ls_ref, hidt_ref, *,
             BT, NV, VC, V):
    j = pl.program_id(1)

    @pl.when(j == 0)
    def _stats():
        hidt_ref[...] = hid_ref[...].T
        hid_t = hidt_ref[...]
        m = jnp.full((1, BT), -1e30, jnp.float32)
        s = jnp.zeros((1, BT), jnp.float32)
        for k in range(NV):
            lo = k * VC
            hi = min(lo + VC, V)
            wc = wtb_all_ref[:, lo:hi]
            l = lax.dot_general(wc, hid_t, _TN,
                                preferred_element_type=jnp.float32)
            cm = jnp.max(l, axis=0, keepdims=True)
            m2 = jnp.maximum(m, cm)
            s = s * jnp.exp(m - m2) + jnp.sum(
                jnp.exp(l - m2), axis=0, keepdims=True
            )
            m = m2
        ls_ref[...] = m + jnp.log(s)

    l = lax.dot_general(wtb_blk_ref[...], hidt_ref[...], _TN,
                        preferred_element_type=jnp.float32)
    out_ref[...] = l - ls_ref[...]


@functools.lru_cache(maxsize=None)
def _make_tc_logsoftmax(B, V, D, BT, VC):
    NV = -(-V // VC)
    DK = D + 1
    body = functools.partial(_tc_body, BT=BT, NV=NV, VC=VC, V=V)
    return pl.pallas_call(
        body,
        grid=(B // BT, NV),
        in_specs=[
            pl.BlockSpec((BT, DK), lambda i, j: (i, 0)),
            pl.BlockSpec((DK, V), lambda i, j: (0, 0)),
            pl.BlockSpec((DK, VC), lambda i, j: (0, j)),
        ],
        out_specs=pl.BlockSpec((VC, BT), lambda i, j: (j, i)),
        out_shape=jax.ShapeDtypeStruct((V, B), jnp.float32),
        scratch_shapes=[
            pltpu.VMEM((1, BT), jnp.float32),
            pltpu.VMEM((DK, BT), jnp.float32),
        ],
    )


def kernel(center_word_index, emb_table, W, b):
    V, D = emb_table.shape
    (B,) = center_word_index.shape
    BT = 256
    VC = 2048
    NV = -(-V // VC)
    VPAD = NV * VC

    idx = center_word_index.astype(jnp.int32)
    hidden = _make_sc_gather(V, D, B)(emb_table, idx)

    wtb = jnp.concatenate([W.T, b[None, :]], axis=0)
    hid1 = jnp.concatenate(
        [hidden, jnp.ones((B, 1), jnp.float32)], axis=1
    )

    out_t = _make_tc_logsoftmax(B, V, D, BT, VC)(hid1, wtb, wtb)
    return out_t.T


# speedup vs baseline: 2.4284x; 1.1265x over previous
import functools

import jax
import jax.numpy as jnp
from jax import lax
from jax.experimental import pallas as pl
from jax.experimental.pallas import tpu as pltpu
from jax.experimental.pallas import tpu_sc as plsc


@functools.lru_cache(maxsize=None)
def _make_sc_gather(V, D, B):
    info = plsc.get_sparse_core_info()
    NC, NS, L = info.num_cores, info.num_subcores, info.num_lanes
    NW = NC * NS
    assert D % L == 0 and B % (8 * NW) == 0
    b_per_w = B // NW
    mesh = plsc.VectorSubcoreMesh(core_axis_name="c", subcore_axis_name="s")

    @functools.partial(
        pl.kernel,
        mesh=mesh,
        out_type=jax.ShapeDtypeStruct((B, D), jnp.float32),
        scratch_types=[
            pltpu.VMEM((b_per_w,), jnp.int32),
            pltpu.VMEM((b_per_w, D), jnp.float32),
            pltpu.SemaphoreType.DMA,
        ],
        compiler_params=pltpu.CompilerParams(use_tc_tiling_on_sc=False),
    )
    def sc_gather(table_hbm, idx_hbm, out_hbm, idx_v, rows_v, sem):
        wid = lax.axis_index("s") * NC + lax.axis_index("c")
        base = wid * b_per_w
        pltpu.sync_copy(idx_hbm.at[pl.ds(base, b_per_w)], idx_v)
        pltpu.async_copy(table_hbm.at[idx_v], rows_v, sem).wait()
        pltpu.sync_copy(rows_v, out_hbm.at[pl.ds(base, b_per_w)])

    return sc_gather


_TN = (((0,), (0,)), ((), ()))


def _tc_body(hid_ref, wtb_ref, out_ref, m_ref, s_ref, hidt_ref, *,
             BT, NV, NB, VC, DK):
    i = pl.program_id(0)
    j = pl.program_id(1)
    a = i % 2

    @pl.when((j == 0) & (i < NB))
    def _init():
        hidt_ref[a] = hid_ref[...].T
        m_ref[...] = jnp.full((1, BT), -1e30, jnp.float32)
        s_ref[...] = jnp.zeros((1, BT), jnp.float32)

    @pl.when(i < NB)
    def _acc():
        l = lax.dot_general(wtb_ref[...], hidt_ref[a], _TN,
                            preferred_element_type=jnp.float32)
        m = m_ref[...]
        m2 = jnp.maximum(m, jnp.max(l, axis=0, keepdims=True))
        s_ref[...] = s_ref[...] * jnp.exp(m - m2) + jnp.sum(
            jnp.exp(l - m2), axis=0, keepdims=True
        )
        m_ref[...] = m2

    @pl.when((j == NV - 1) & (i < NB))
    def _fin():
        hidt_ref[a, DK - 1 : DK, :] = -(m_ref[...] + jnp.log(s_ref[...]))

    @pl.when(i > 0)
    def _write():
        out_ref[...] = lax.dot_general(wtb_ref[...], hidt_ref[1 - a], _TN,
                                       preferred_element_type=jnp.float32)


@functools.lru_cache(maxsize=None)
def _make_tc_logsoftmax(B, V, D, BT, VC):
    NV = -(-V // VC)
    NB = B // BT
    DK = D + 2
    body = functools.partial(_tc_body, BT=BT, NV=NV, NB=NB, VC=VC, DK=DK)
    return pl.pallas_call(
        body,
        grid=(NB + 1, NV),
        in_specs=[
            pl.BlockSpec((BT, DK), lambda i, j: (jnp.minimum(i, NB - 1), 0)),
            pl.BlockSpec((DK, VC), lambda i, j: (0, j)),
        ],
        out_specs=pl.BlockSpec(
            (VC, BT),
            lambda i, j: (jnp.where(i == 0, 0, j), jnp.maximum(i - 1, 0)),
        ),
        out_shape=jax.ShapeDtypeStruct((V, B), jnp.float32),
        scratch_shapes=[
            pltpu.VMEM((1, BT), jnp.float32),
            pltpu.VMEM((1, BT), jnp.float32),
            pltpu.VMEM((2, DK, BT), jnp.float32),
        ],
    )


def kernel(center_word_index, emb_table, W, b):
    V, D = emb_table.shape
    (B,) = center_word_index.shape
    BT = 256
    VC = 2048
    NV = -(-V // VC)
    VPAD = NV * VC

    idx = center_word_index.astype(jnp.int32)
    hidden = _make_sc_gather(V, D, B)(emb_table, idx)

    wt_p = jnp.pad(W.T, ((0, 0), (0, VPAD - V)))
    b_p = jnp.pad(b, (0, VPAD - V), constant_values=-1e30)
    wtb = jnp.concatenate(
        [wt_p, b_p[None, :], jnp.ones((1, VPAD), jnp.float32)], axis=0
    )
    hid1 = jnp.concatenate(
        [hidden, jnp.ones((B, 1), jnp.float32), jnp.zeros((B, 1), jnp.float32)],
        axis=1,
    )

    out_t = _make_tc_logsoftmax(B, V, D, BT, VC)(hid1, wtb)
    return out_t.T


# speedup vs baseline: 3.1965x; 1.3163x over previous
import functools

import jax
import jax.numpy as jnp
from jax import lax
from jax.experimental import pallas as pl
from jax.experimental.pallas import tpu as pltpu
from jax.experimental.pallas import tpu_sc as plsc


@functools.lru_cache(maxsize=None)
def _make_sc_gather(V, D, B):
    info = plsc.get_sparse_core_info()
    NC, NS, L = info.num_cores, info.num_subcores, info.num_lanes
    NW = NC * NS
    assert D % L == 0 and B % (8 * NW) == 0
    b_per_w = B // NW
    mesh = plsc.VectorSubcoreMesh(core_axis_name="c", subcore_axis_name="s")

    @functools.partial(
        pl.kernel,
        mesh=mesh,
        out_type=jax.ShapeDtypeStruct((B, D), jnp.float32),
        scratch_types=[
            pltpu.VMEM((b_per_w,), jnp.int32),
            pltpu.VMEM((b_per_w, D), jnp.float32),
            pltpu.SemaphoreType.DMA,
        ],
        compiler_params=pltpu.CompilerParams(use_tc_tiling_on_sc=False),
    )
    def sc_gather(table_hbm, idx_hbm, out_hbm, idx_v, rows_v, sem):
        wid = lax.axis_index("s") * NC + lax.axis_index("c")
        base = wid * b_per_w
        pltpu.sync_copy(idx_hbm.at[pl.ds(base, b_per_w)], idx_v)
        pltpu.async_copy(table_hbm.at[idx_v], rows_v, sem).wait()
        pltpu.sync_copy(rows_v, out_hbm.at[pl.ds(base, b_per_w)])

    return sc_gather


_TN = (((0,), (0,)), ((), ()))


def _tc_body(hid_ref, wtb_ref, out_ref, m_ref, s_ref, hidt_ref, *,
             BT, NV, NB, VC, DK):
    i = pl.program_id(0)
    j = pl.program_id(1)
    a = i % 2

    @pl.when(j == 0)
    def _init():
        hidt_ref[a] = hid_ref[...].T
        m_ref[...] = jnp.full((1, BT), -1e30, jnp.float32)
        s_ref[...] = jnp.zeros((1, BT), jnp.float32)

    l = lax.dot_general(wtb_ref[...], hidt_ref[a], _TN,
                        preferred_element_type=jnp.float32)
    out_ref[...] = lax.dot_general(wtb_ref[...], hidt_ref[1 - a], _TN,
                                   preferred_element_type=jnp.float32)
    m = m_ref[...]
    m2 = jnp.maximum(m, jnp.max(l, axis=0, keepdims=True))
    s_ref[...] = s_ref[...] * jnp.exp(m - m2) + jnp.sum(
        jnp.exp(l - m2), axis=0, keepdims=True
    )
    m_ref[...] = m2

    @pl.when(j == NV - 1)
    def _fin():
        hidt_ref[a, DK - 1 : DK, :] = -(m_ref[...] + jnp.log(s_ref[...]))


@functools.lru_cache(maxsize=None)
def _make_tc_logsoftmax(B, V, D, BT, VC):
    NV = -(-V // VC)
    NB = B // BT
    DK = D + 2
    body = functools.partial(_tc_body, BT=BT, NV=NV, NB=NB, VC=VC, DK=DK)
    return pl.pallas_call(
        body,
        grid=(NB + 1, NV),
        in_specs=[
            pl.BlockSpec((BT, DK), lambda i, j: (jnp.minimum(i, NB - 1), 0)),
            pl.BlockSpec((DK, VC), lambda i, j: (0, j)),
        ],
        out_specs=pl.BlockSpec(
            (VC, BT),
            lambda i, j: (jnp.where(i == 0, 0, j), jnp.maximum(i - 1, 0)),
        ),
        out_shape=jax.ShapeDtypeStruct((V, B), jnp.float32),
        scratch_shapes=[
            pltpu.VMEM((1, BT), jnp.float32),
            pltpu.VMEM((1, BT), jnp.float32),
            pltpu.VMEM((2, DK, BT), jnp.float32),
        ],
    )


def kernel(center_word_index, emb_table, W, b):
    V, D = emb_table.shape
    (B,) = center_word_index.shape
    BT = 256
    VC = 4096
    NV = -(-V // VC)
    VPAD = NV * VC

    idx = center_word_index.astype(jnp.int32)
    hidden = _make_sc_gather(V, D, B)(emb_table, idx)

    wt_p = jnp.pad(W.T, ((0, 0), (0, VPAD - V)))
    b_p = jnp.pad(b, (0, VPAD - V), constant_values=-1e30)
    wtb = jnp.concatenate(
        [wt_p, b_p[None, :], jnp.ones((1, VPAD), jnp.float32)], axis=0
    )
    hid1 = jnp.concatenate(
        [hidden, jnp.ones((B, 1), jnp.float32), jnp.zeros((B, 1), jnp.float32)],
        axis=1,
    )

    out_t = _make_tc_logsoftmax(B, V, D, BT, VC)(hid1, wtb)
    return out_t.T


# speedup vs baseline: 3.5698x; 1.1168x over previous
import functools

import jax
import jax.numpy as jnp
from jax import lax
from jax.experimental import pallas as pl
from jax.experimental.pallas import tpu as pltpu
from jax.experimental.pallas import tpu_sc as plsc


@functools.lru_cache(maxsize=None)
def _make_sc_gather(V, D, B):
    info = plsc.get_sparse_core_info()
    NC, NS, L = info.num_cores, info.num_subcores, info.num_lanes
    NW = NC * NS
    assert D % L == 0 and B % (8 * NW) == 0
    b_per_w = B // NW
    mesh = plsc.VectorSubcoreMesh(core_axis_name="c", subcore_axis_name="s")

    @functools.partial(
        pl.kernel,
        mesh=mesh,
        out_type=jax.ShapeDtypeStruct((B, D), jnp.float32),
        scratch_types=[
            pltpu.VMEM((b_per_w,), jnp.int32),
            pltpu.VMEM((b_per_w, D), jnp.float32),
            pltpu.SemaphoreType.DMA,
        ],
        compiler_params=pltpu.CompilerParams(use_tc_tiling_on_sc=False),
    )
    def sc_gather(table_hbm, idx_hbm, out_hbm, idx_v, rows_v, sem):
        wid = lax.axis_index("s") * NC + lax.axis_index("c")
        base = wid * b_per_w
        pltpu.sync_copy(idx_hbm.at[pl.ds(base, b_per_w)], idx_v)
        pltpu.async_copy(table_hbm.at[idx_v], rows_v, sem).wait()
        pltpu.sync_copy(rows_v, out_hbm.at[pl.ds(base, b_per_w)])

    return sc_gather


_TN = (((0,), (0,)), ((), ()))


def _tc_body(hid_ref, wtb_ref, out_ref, m_ref, s_ref, hidt_ref, *,
             BT, NV, NB, VC, DK):
    i = pl.program_id(0)
    j = pl.program_id(1)
    a = i % 2

    @pl.when(j == 0)
    def _init():
        hidt_ref[a] = hid_ref[...].T
        m_ref[...] = jnp.full((1, BT), -1e30, jnp.float32)
        s_ref[...] = jnp.zeros((1, BT), jnp.float32)

    l = lax.dot_general(wtb_ref[...], hidt_ref[a], _TN,
                        preferred_element_type=jnp.float32)
    out_ref[...] = lax.dot_general(wtb_ref[...], hidt_ref[1 - a], _TN,
                                   preferred_element_type=jnp.float32)
    m = m_ref[...]
    m2 = jnp.maximum(m, jnp.max(l, axis=0, keepdims=True))
    s_ref[...] = s_ref[...] * jnp.exp(m - m2) + jnp.sum(
        jnp.exp(l - m2), axis=0, keepdims=True
    )
    m_ref[...] = m2

    @pl.when(j == NV - 1)
    def _fin():
        hidt_ref[a, DK - 1 : DK, :] = -(m_ref[...] + jnp.log(s_ref[...]))


@functools.lru_cache(maxsize=None)
def _make_tc_logsoftmax(B, V, D, BT, VC):
    NV = -(-V // VC)
    NB = B // BT
    DK = D + 2
    body = functools.partial(_tc_body, BT=BT, NV=NV, NB=NB, VC=VC, DK=DK)
    return pl.pallas_call(
        body,
        grid=(NB + 1, NV),
        in_specs=[
            pl.BlockSpec((BT, DK), lambda i, j: (jnp.minimum(i, NB - 1), 0)),
            pl.BlockSpec((DK, VC), lambda i, j: (0, j)),
        ],
        out_specs=pl.BlockSpec(
            (VC, BT),
            lambda i, j: (jnp.where(i == 0, 0, j), jnp.maximum(i - 1, 0)),
        ),
        out_shape=jax.ShapeDtypeStruct((V, B), jnp.float32),
        scratch_shapes=[
            pltpu.VMEM((1, BT), jnp.float32),
            pltpu.VMEM((1, BT), jnp.float32),
            pltpu.VMEM((2, DK, BT), jnp.float32),
        ],
    )


def kernel(center_word_index, emb_table, W, b):
    V, D = emb_table.shape
    (B,) = center_word_index.shape
    BT = 512
    VC = 4096
    NV = -(-V // VC)
    VPAD = NV * VC

    idx = center_word_index.astype(jnp.int32)
    hidden = _make_sc_gather(V, D, B)(emb_table, idx)

    wt_p = jnp.pad(W.T, ((0, 0), (0, VPAD - V)))
    b_p = jnp.pad(b, (0, VPAD - V), constant_values=-1e30)
    wtb = jnp.concatenate(
        [wt_p, b_p[None, :], jnp.ones((1, VPAD), jnp.float32)], axis=0
    )
    hid1 = jnp.concatenate(
        [hidden, jnp.ones((B, 1), jnp.float32), jnp.zeros((B, 1), jnp.float32)],
        axis=1,
    )

    out_t = _make_tc_logsoftmax(B, V, D, BT, VC)(hid1, wtb)
    return out_t.T
